# Initial kernel scaffold; baseline (speedup 1.0000x reference)
import jax
import jax.numpy as jnp
from jax.experimental import pallas as pl


def kernel(x, edge_index, edge_attr, lin1_W, lin1_b, root1, bias1, lin2_W, lin2_b, root2, bias2):
    raise NotImplementedError("write your pallas kernel here")


# speedup vs baseline: 3.5977x; 3.5977x over previous
import functools

import jax
import jax.numpy as jnp
from jax import lax
from jax.experimental import pallas as pl
from jax.experimental.pallas import tpu as pltpu
from jax.experimental.pallas import tpu_sc as plsc

NC, NS, L = 2, 16, 16
NW = NC * NS
N = 50000
NPAD = 50176
SPAN = NPAD // NW
E = 1600000
GROUPS = 4
TPG = NW // GROUPS
EPT = E // TPG
C = 1600
NCHUNK = EPT // C


def _mesh():
    return plsc.VectorSubcoreMesh(core_axis_name="c", subcore_axis_name="s")


def _wid():
    return lax.axis_index("s") * NC + lax.axis_index("c")


def _bcast_param(par_v, idx):
    return plsc.load_gather(par_v, [jnp.zeros((L,), jnp.int32) + idx])


def _edge_pass(table, src, dst, attr_flat, params, grouped_table):
    @functools.partial(
        pl.kernel,
        mesh=_mesh(),
        compiler_params=pltpu.CompilerParams(needs_layout_passes=False),
        out_type=jax.ShapeDtypeStruct((NW * NPAD,), jnp.float32),
        scratch_types=[
            pltpu.VMEM((NPAD,), jnp.float32),
            pltpu.VMEM((NPAD,), jnp.float32),
            pltpu.VMEM((C,), jnp.int32),
            pltpu.VMEM((C,), jnp.int32),
            pltpu.VMEM((3 * C,), jnp.float32),
            pltpu.VMEM((L,), jnp.float32),
        ],
    )
    def body(table_h, src_h, dst_h, attr_h, par_h, out_h,
             tab_v, acc_v, src_v, dst_v, attr_v, par_v):
        wid = _wid()
        g = wid // TPG
        t = wid % TPG
        pltpu.sync_copy(par_h, par_v)
        if grouped_table:
            pltpu.sync_copy(table_h.at[pl.ds(g * NPAD, NPAD)], tab_v)
        else:
            pltpu.sync_copy(table_h, tab_v)

        zeros16 = jnp.zeros((L,), jnp.float32)

        def zero_body(i, c):
            acc_v[pl.ds(i * L, L)] = zeros16
            return c

        lax.fori_loop(0, NPAD // L, zero_body, 0)

        wk0 = _bcast_param(par_v, 3 * g)
        wk1 = _bcast_param(par_v, 3 * g + 1)
        wk2 = _bcast_param(par_v, 3 * g + 2)
        bk = _bcast_param(par_v, 12 + g)
        iota = lax.iota(jnp.int32, L)
        ebase = t * EPT

        def chunk_body(ci, c):
            base = ebase + ci * C
            pltpu.sync_copy(src_h.at[pl.ds(base, C)], src_v)
            pltpu.sync_copy(dst_h.at[pl.ds(base, C)], dst_v)
            pltpu.sync_copy(attr_h.at[pl.ds(3 * base, 3 * C)], attr_v)

            def grp_body(gi, cc):
                off = gi * L
                ei3 = (off + iota) * 3
                a0 = plsc.load_gather(attr_v, [ei3])
                a1 = plsc.load_gather(attr_v, [ei3 + 1])
                a2 = plsc.load_gather(attr_v, [ei3 + 2])
                w = jnp.maximum(a0 * wk0 + a1 * wk1 + a2 * wk2 + bk, 0.0)
                sv = src_v[pl.ds(off, L)]
                dv = dst_v[pl.ds(off, L)]
                xs = plsc.load_gather(tab_v, [sv])
                plsc.addupdate_scatter(acc_v, [dv], xs * w)
                return cc

            lax.fori_loop(0, C // L, grp_body, 0)
            return c

        lax.fori_loop(0, NCHUNK, chunk_body, 0)
        pltpu.sync_copy(acc_v, out_h.at[pl.ds(wid * NPAD, NPAD)])

    return body(table, src, dst, attr_flat, params)


def _build_x1(partials, x_pad, params):
    @functools.partial(
        pl.kernel,
        mesh=_mesh(),
        compiler_params=pltpu.CompilerParams(needs_layout_passes=False),
        out_type=jax.ShapeDtypeStruct((GROUPS * NPAD,), jnp.float32),
        scratch_types=[
            pltpu.VMEM((NW * SPAN,), jnp.float32),
            pltpu.VMEM((SPAN,), jnp.float32),
            pltpu.VMEM((SPAN,), jnp.float32),
            pltpu.VMEM((L,), jnp.float32),
        ],
    )
    def body(par_h, x_h, prm_h, out_h, pbuf_v, xb_v, cb_v, prm_v):
        wid = _wid()
        nb = wid * SPAN
        pltpu.sync_copy(prm_h, prm_v)
        for tt in range(NW):
            pltpu.sync_copy(par_h.at[pl.ds(tt * NPAD + nb, SPAN)],
                            pbuf_v.at[pl.ds(tt * SPAN, SPAN)])
        pltpu.sync_copy(x_h.at[pl.ds(nb, SPAN)], xb_v)
        for o in range(GROUPS):
            ro = _bcast_param(prm_v, o)
            bo = _bcast_param(prm_v, 4 + o)

            def strip(j, c, o=o, ro=ro, bo=bo):
                off = j * L
                s = jnp.zeros((L,), jnp.float32)
                for tt in range(o * TPG, (o + 1) * TPG):
                    s = s + pbuf_v[pl.ds(tt * SPAN + off, L)]
                xv = xb_v[pl.ds(off, L)]
                cb_v[pl.ds(off, L)] = jnp.maximum(s + xv * ro + bo, 0.0)
                return c

            lax.fori_loop(0, SPAN // L, strip, 0)
            pltpu.sync_copy(cb_v, out_h.at[pl.ds(o * NPAD + nb, SPAN)])

    return body(partials, x_pad, params)


def _build_x2(partials, x1t, params):
    @functools.partial(
        pl.kernel,
        mesh=_mesh(),
        compiler_params=pltpu.CompilerParams(needs_layout_passes=False),
        out_type=jax.ShapeDtypeStruct((NPAD,), jnp.float32),
        scratch_types=[
            pltpu.VMEM((NW * SPAN,), jnp.float32),
            pltpu.VMEM((GROUPS * SPAN,), jnp.float32),
            pltpu.VMEM((SPAN,), jnp.float32),
            pltpu.VMEM((L,), jnp.float32),
        ],
    )
    def body(par_h, x1_h, prm_h, out_h, pbuf_v, x1b_v, cb_v, prm_v):
        wid = _wid()
        nb = wid * SPAN
        pltpu.sync_copy(prm_h, prm_v)
        for tt in range(NW):
            pltpu.sync_copy(par_h.at[pl.ds(tt * NPAD + nb, SPAN)],
                            pbuf_v.at[pl.ds(tt * SPAN, SPAN)])
        for i in range(GROUPS):
            pltpu.sync_copy(x1_h.at[pl.ds(i * NPAD + nb, SPAN)],
                            x1b_v.at[pl.ds(i * SPAN, SPAN)])
        rv = [_bcast_param(prm_v, i) for i in range(GROUPS)]
        bv = _bcast_param(prm_v, GROUPS)

        def strip(j, c):
            off = j * L
            s = jnp.zeros((L,), jnp.float32)
            for tt in range(NW):
                s = s + pbuf_v[pl.ds(tt * SPAN + off, L)]
            for i in range(GROUPS):
                s = s + x1b_v[pl.ds(i * SPAN + off, L)] * rv[i]
            z = s + bv
            cb_v[pl.ds(off, L)] = 1.0 / (1.0 + jnp.exp(-z))
            return c

        lax.fori_loop(0, SPAN // L, strip, 0)
        pltpu.sync_copy(cb_v, out_h.at[pl.ds(nb, SPAN)])

    return body(partials, x1t, params)


def kernel(x, edge_index, edge_attr,
           lin1_W, lin1_b, root1, bias1,
           lin2_W, lin2_b, root2, bias2):
    ei = edge_index.astype(jnp.int32)
    src = ei[0]
    dst = ei[1]
    attr_flat = edge_attr.reshape(-1)
    x_pad = jnp.pad(x, (0, NPAD - N))
    p1 = jnp.concatenate([lin1_W.reshape(-1), lin1_b])
    p2 = jnp.concatenate([root1.reshape(-1), bias1,
                          jnp.zeros((8,), jnp.float32)])
    p3 = jnp.concatenate([lin2_W.reshape(-1), lin2_b])
    p4 = jnp.concatenate([root2.reshape(-1), bias2,
                          jnp.zeros((11,), jnp.float32)])
    parts1 = _edge_pass(x_pad, src, dst, attr_flat, p1, grouped_table=False)
    x1t = _build_x1(parts1, x_pad, p2)
    parts2 = _edge_pass(x1t, src, dst, attr_flat, p3, grouped_table=True)
    x2 = _build_x2(parts2, x1t, p4)
    return x2[:N].reshape(N, 1)


# speedup vs baseline: 13.4721x; 3.7446x over previous
import functools

import jax
import jax.numpy as jnp
from jax import lax
from jax.experimental import pallas as pl
from jax.experimental.pallas import tpu as pltpu
from jax.experimental.pallas import tpu_sc as plsc

NC, NS, L = 2, 16, 16
NW = NC * NS
N = 50000
NPAD = 50176
SPAN = NPAD // NW
E = 1600000
GROUPS = 4
TPG = NW // GROUPS
EPT = E // TPG
C = 1600
NCHUNK = EPT // C


def _mesh():
    return plsc.VectorSubcoreMesh(core_axis_name="c", subcore_axis_name="s")


def _wid():
    return lax.axis_index("s") * NC + lax.axis_index("c")


def _bcast_param(par_v, idx):
    return plsc.load_gather(par_v, [jnp.zeros((L,), jnp.int32) + idx])


def _edge_pass(table, src, dst, attr_flat, params, grouped_table):
    @functools.partial(
        pl.kernel,
        mesh=_mesh(),
        compiler_params=pltpu.CompilerParams(needs_layout_passes=False),
        out_type=jax.ShapeDtypeStruct((NW * NPAD,), jnp.float32),
        scratch_types=[
            pltpu.VMEM((NPAD,), jnp.float32),
            pltpu.VMEM((NPAD,), jnp.float32),
            pltpu.VMEM((C,), jnp.int32),
            pltpu.VMEM((C,), jnp.int32),
            pltpu.VMEM((3 * C,), jnp.float32),
            pltpu.VMEM((L,), jnp.float32),
        ],
    )
    def body(table_h, src_h, dst_h, attr_h, par_h, out_h,
             tab_v, acc_v, src_v, dst_v, attr_v, par_v):
        wid = _wid()
        g = wid // TPG
        t = wid % TPG
        pltpu.sync_copy(par_h, par_v)
        if grouped_table:
            pltpu.sync_copy(table_h.at[pl.ds(g * NPAD, NPAD)], tab_v)
        else:
            pltpu.sync_copy(table_h, tab_v)

        zeros16 = jnp.zeros((L,), jnp.float32)

        def zero_body(i, c):
            acc_v[pl.ds(i * L, L)] = zeros16
            return c

        lax.fori_loop(0, NPAD // L, zero_body, 0)

        wk0 = _bcast_param(par_v, 3 * g)
        wk1 = _bcast_param(par_v, 3 * g + 1)
        wk2 = _bcast_param(par_v, 3 * g + 2)
        bk = _bcast_param(par_v, 12 + g)
        iota = lax.iota(jnp.int32, L)
        ebase = t * EPT

        def chunk_body(ci, c):
            base = ebase + ci * C
            pltpu.sync_copy(src_h.at[pl.ds(base, C)], src_v)
            pltpu.sync_copy(dst_h.at[pl.ds(base, C)], dst_v)
            for k in range(3):
                pltpu.sync_copy(attr_h.at[pl.ds(k * E + base, C)],
                                attr_v.at[pl.ds(k * C, C)])

            def grp_body(gi, cc):
                off = gi * L
                a0 = attr_v[pl.ds(off, L)]
                a1 = attr_v[pl.ds(C + off, L)]
                a2 = attr_v[pl.ds(2 * C + off, L)]
                w = jnp.maximum(a0 * wk0 + a1 * wk1 + a2 * wk2 + bk, 0.0)
                sv = src_v[pl.ds(off, L)]
                dv = dst_v[pl.ds(off, L)]
                xs = plsc.load_gather(tab_v, [sv])
                plsc.addupdate_scatter(acc_v, [dv], xs * w)
                return cc

            lax.fori_loop(0, C // L, grp_body, 0)
            return c

        lax.fori_loop(0, NCHUNK, chunk_body, 0)
        pltpu.sync_copy(acc_v, out_h.at[pl.ds(wid * NPAD, NPAD)])

    return body(table, src, dst, attr_flat, params)


def _build_x1(partials, x_pad, params):
    @functools.partial(
        pl.kernel,
        mesh=_mesh(),
        compiler_params=pltpu.CompilerParams(needs_layout_passes=False),
        out_type=jax.ShapeDtypeStruct((GROUPS * NPAD,), jnp.float32),
        scratch_types=[
            pltpu.VMEM((NW * SPAN,), jnp.float32),
            pltpu.VMEM((SPAN,), jnp.float32),
            pltpu.VMEM((SPAN,), jnp.float32),
            pltpu.VMEM((L,), jnp.float32),
        ],
    )
    def body(par_h, x_h, prm_h, out_h, pbuf_v, xb_v, cb_v, prm_v):
        wid = _wid()
        nb = wid * SPAN
        pltpu.sync_copy(prm_h, prm_v)
        for tt in range(NW):
            pltpu.sync_copy(par_h.at[pl.ds(tt * NPAD + nb, SPAN)],
                            pbuf_v.at[pl.ds(tt * SPAN, SPAN)])
        pltpu.sync_copy(x_h.at[pl.ds(nb, SPAN)], xb_v)
        for o in range(GROUPS):
            ro = _bcast_param(prm_v, o)
            bo = _bcast_param(prm_v, 4 + o)

            def strip(j, c, o=o, ro=ro, bo=bo):
                off = j * L
                s = jnp.zeros((L,), jnp.float32)
                for tt in range(o * TPG, (o + 1) * TPG):
                    s = s + pbuf_v[pl.ds(tt * SPAN + off, L)]
                xv = xb_v[pl.ds(off, L)]
                cb_v[pl.ds(off, L)] = jnp.maximum(s + xv * ro + bo, 0.0)
                return c

            lax.fori_loop(0, SPAN // L, strip, 0)
            pltpu.sync_copy(cb_v, out_h.at[pl.ds(o * NPAD + nb, SPAN)])

    return body(partials, x_pad, params)


def _build_x2(partials, x1t, params):
    @functools.partial(
        pl.kernel,
        mesh=_mesh(),
        compiler_params=pltpu.CompilerParams(needs_layout_passes=False),
        out_type=jax.ShapeDtypeStruct((NPAD,), jnp.float32),
        scratch_types=[
            pltpu.VMEM((NW * SPAN,), jnp.float32),
            pltpu.VMEM((GROUPS * SPAN,), jnp.float32),
            pltpu.VMEM((SPAN,), jnp.float32),
            pltpu.VMEM((L,), jnp.float32),
        ],
    )
    def body(par_h, x1_h, prm_h, out_h, pbuf_v, x1b_v, cb_v, prm_v):
        wid = _wid()
        nb = wid * SPAN
        pltpu.sync_copy(prm_h, prm_v)
        for tt in range(NW):
            pltpu.sync_copy(par_h.at[pl.ds(tt * NPAD + nb, SPAN)],
                            pbuf_v.at[pl.ds(tt * SPAN, SPAN)])
        for i in range(GROUPS):
            pltpu.sync_copy(x1_h.at[pl.ds(i * NPAD + nb, SPAN)],
                            x1b_v.at[pl.ds(i * SPAN, SPAN)])
        rv = [_bcast_param(prm_v, i) for i in range(GROUPS)]
        bv = _bcast_param(prm_v, GROUPS)

        def strip(j, c):
            off = j * L
            s = jnp.zeros((L,), jnp.float32)
            for tt in range(NW):
                s = s + pbuf_v[pl.ds(tt * SPAN + off, L)]
            for i in range(GROUPS):
                s = s + x1b_v[pl.ds(i * SPAN + off, L)] * rv[i]
            z = s + bv
            cb_v[pl.ds(off, L)] = 1.0 / (1.0 + jnp.exp(-z))
            return c

        lax.fori_loop(0, SPAN // L, strip, 0)
        pltpu.sync_copy(cb_v, out_h.at[pl.ds(nb, SPAN)])

    return body(partials, x1t, params)


def kernel(x, edge_index, edge_attr,
           lin1_W, lin1_b, root1, bias1,
           lin2_W, lin2_b, root2, bias2):
    ei = edge_index.astype(jnp.int32)
    src = ei[0]
    dst = ei[1]
    attr_flat = edge_attr.T.reshape(-1)
    x_pad = jnp.pad(x, (0, NPAD - N))
    p1 = jnp.concatenate([lin1_W.reshape(-1), lin1_b])
    p2 = jnp.concatenate([root1.reshape(-1), bias1,
                          jnp.zeros((8,), jnp.float32)])
    p3 = jnp.concatenate([lin2_W.reshape(-1), lin2_b])
    p4 = jnp.concatenate([root2.reshape(-1), bias2,
                          jnp.zeros((11,), jnp.float32)])
    parts1 = _edge_pass(x_pad, src, dst, attr_flat, p1, grouped_table=False)
    x1t = _build_x1(parts1, x_pad, p2)
    parts2 = _edge_pass(x1t, src, dst, attr_flat, p3, grouped_table=True)
    x2 = _build_x2(parts2, x1t, p4)
    return x2[:N].reshape(N, 1)


# speedup vs baseline: 25.3121x; 1.8788x over previous
import functools

import jax
import jax.numpy as jnp
from jax import lax
from jax.experimental import pallas as pl
from jax.experimental.pallas import tpu as pltpu
from jax.experimental.pallas import tpu_sc as plsc

NC, NS, L = 2, 16, 16
NW = NC * NS
N = 50000
NPAD = 50176
SPAN = NPAD // NW
E = 1600000
GROUPS = 4
TPG = NW // GROUPS
EPT = E // TPG
C = 2000
NCHUNK = EPT // C


def _mesh():
    return plsc.VectorSubcoreMesh(core_axis_name="c", subcore_axis_name="s")


def _wid():
    return lax.axis_index("s") * NC + lax.axis_index("c")


def _prow(par_v, row):
    return par_v[pl.ds(row * L, L)]


def _edge_pass(table, src, dst, attr_flat, params, grouped_table):
    @functools.partial(
        pl.kernel,
        mesh=_mesh(),
        compiler_params=pltpu.CompilerParams(needs_layout_passes=False),
        out_type=jax.ShapeDtypeStruct((NW * NPAD,), jnp.float32),
        scratch_types=[
            pltpu.VMEM((NPAD,), jnp.float32),
            pltpu.VMEM((NPAD,), jnp.float32),
            pltpu.VMEM((2 * C,), jnp.int32),
            pltpu.VMEM((2 * C,), jnp.int32),
            pltpu.VMEM((2 * 3 * C,), jnp.float32),
            pltpu.VMEM((16 * L,), jnp.float32),
            pltpu.SemaphoreType.DMA,
            pltpu.SemaphoreType.DMA,
        ],
    )
    def body(table_h, src_h, dst_h, attr_h, par_h, out_h,
             tab_v, acc_v, src_v, dst_v, attr_v, par_v, sem0, sem1):
        wid = _wid()
        g = wid // TPG
        t = wid % TPG
        pltpu.sync_copy(par_h, par_v)
        if grouped_table:
            pltpu.sync_copy(table_h.at[pl.ds(g * NPAD, NPAD)], tab_v)
        else:
            pltpu.sync_copy(table_h, tab_v)

        zeros16 = jnp.zeros((L,), jnp.float32)

        def zero_body(i, c):
            acc_v[pl.ds(i * L, L)] = zeros16
            return c

        lax.fori_loop(0, NPAD // L, zero_body, 0)

        wk0 = _prow(par_v, 3 * g)
        wk1 = _prow(par_v, 3 * g + 1)
        wk2 = _prow(par_v, 3 * g + 2)
        bk = _prow(par_v, 12 + g)
        ebase = t * EPT
        sems = (sem0, sem1)

        def chunk_copies(ci, slot):
            base = ebase + ci * C
            so = slot * C
            ao = slot * 3 * C
            return [
                (src_h.at[pl.ds(base, C)], src_v.at[pl.ds(so, C)]),
                (dst_h.at[pl.ds(base, C)], dst_v.at[pl.ds(so, C)]),
                (attr_h.at[pl.ds(base, C)], attr_v.at[pl.ds(ao, C)]),
                (attr_h.at[pl.ds(E + base, C)], attr_v.at[pl.ds(ao + C, C)]),
                (attr_h.at[pl.ds(2 * E + base, C)],
                 attr_v.at[pl.ds(ao + 2 * C, C)]),
            ]

        def start(ci, slot):
            for s, d in chunk_copies(ci, slot):
                pltpu.async_copy(s, d, sems[slot])

        def wait(ci, slot):
            for s, d in chunk_copies(ci, slot):
                pltpu.make_async_copy(s, d, sems[slot]).wait()

        def compute(slot):
            so = slot * C
            ao = slot * 3 * C

            def grp_body(gi, cc):
                off = gi * L
                a0 = attr_v[pl.ds(ao + off, L)]
                a1 = attr_v[pl.ds(ao + C + off, L)]
                a2 = attr_v[pl.ds(ao + 2 * C + off, L)]
                w = jnp.maximum(a0 * wk0 + a1 * wk1 + a2 * wk2 + bk, 0.0)
                sv = src_v[pl.ds(so + off, L)]
                dv = dst_v[pl.ds(so + off, L)]
                xs = plsc.load_gather(tab_v, [sv])
                plsc.addupdate_scatter(acc_v, [dv], xs * w)
                return cc

            lax.fori_loop(0, C // L, grp_body, 0)

        start(0, 0)
        start(1, 1)

        def pair_body(j, c):
            c0 = 2 * j
            wait(c0, 0)
            compute(0)
            start(c0 + 2, 0)
            wait(c0 + 1, 1)
            compute(1)
            start(c0 + 3, 1)
            return c

        lax.fori_loop(0, NCHUNK // 2 - 1, pair_body, 0)
        wait(NCHUNK - 2, 0)
        compute(0)
        wait(NCHUNK - 1, 1)
        compute(1)
        pltpu.sync_copy(acc_v, out_h.at[pl.ds(wid * NPAD, NPAD)])

    return body(table, src, dst, attr_flat, params)


def _build_x1(partials, x_pad, params):
    @functools.partial(
        pl.kernel,
        mesh=_mesh(),
        compiler_params=pltpu.CompilerParams(needs_layout_passes=False),
        out_type=jax.ShapeDtypeStruct((GROUPS * NPAD,), jnp.float32),
        scratch_types=[
            pltpu.VMEM((NW * SPAN,), jnp.float32),
            pltpu.VMEM((SPAN,), jnp.float32),
            pltpu.VMEM((GROUPS * SPAN,), jnp.float32),
            pltpu.VMEM((8 * L,), jnp.float32),
        ],
    )
    def body(par_h, x_h, prm_h, out_h, pbuf_v, xb_v, cb_v, prm_v):
        wid = _wid()
        nb = wid * SPAN
        pltpu.sync_copy(prm_h, prm_v)
        for tt in range(NW):
            pltpu.sync_copy(par_h.at[pl.ds(tt * NPAD + nb, SPAN)],
                            pbuf_v.at[pl.ds(tt * SPAN, SPAN)])
        pltpu.sync_copy(x_h.at[pl.ds(nb, SPAN)], xb_v)
        for o in range(GROUPS):
            ro = _prow(prm_v, o)
            bo = _prow(prm_v, 4 + o)

            def strip(j, c, o=o, ro=ro, bo=bo):
                off = j * L
                s = jnp.zeros((L,), jnp.float32)
                for tt in range(o * TPG, (o + 1) * TPG):
                    s = s + pbuf_v[pl.ds(tt * SPAN + off, L)]
                xv = xb_v[pl.ds(off, L)]
                cb_v[pl.ds(o * SPAN + off, L)] = jnp.maximum(s + xv * ro + bo, 0.0)
                return c

            lax.fori_loop(0, SPAN // L, strip, 0)
            pltpu.sync_copy(cb_v.at[pl.ds(o * SPAN, SPAN)],
                            out_h.at[pl.ds(o * NPAD + nb, SPAN)])

    return body(partials, x_pad, params)


def _build_x2(partials, x1t, params):
    @functools.partial(
        pl.kernel,
        mesh=_mesh(),
        compiler_params=pltpu.CompilerParams(needs_layout_passes=False),
        out_type=jax.ShapeDtypeStruct((NPAD,), jnp.float32),
        scratch_types=[
            pltpu.VMEM((NW * SPAN,), jnp.float32),
            pltpu.VMEM((GROUPS * SPAN,), jnp.float32),
            pltpu.VMEM((SPAN,), jnp.float32),
            pltpu.VMEM((5 * L,), jnp.float32),
        ],
    )
    def body(par_h, x1_h, prm_h, out_h, pbuf_v, x1b_v, cb_v, prm_v):
        wid = _wid()
        nb = wid * SPAN
        pltpu.sync_copy(prm_h, prm_v)
        for tt in range(NW):
            pltpu.sync_copy(par_h.at[pl.ds(tt * NPAD + nb, SPAN)],
                            pbuf_v.at[pl.ds(tt * SPAN, SPAN)])
        for i in range(GROUPS):
            pltpu.sync_copy(x1_h.at[pl.ds(i * NPAD + nb, SPAN)],
                            x1b_v.at[pl.ds(i * SPAN, SPAN)])
        rv = [_prow(prm_v, i) for i in range(GROUPS)]
        bv = _prow(prm_v, GROUPS)

        def strip(j, c):
            off = j * L
            s = jnp.zeros((L,), jnp.float32)
            for tt in range(NW):
                s = s + pbuf_v[pl.ds(tt * SPAN + off, L)]
            for i in range(GROUPS):
                s = s + x1b_v[pl.ds(i * SPAN + off, L)] * rv[i]
            z = s + bv
            cb_v[pl.ds(off, L)] = 1.0 / (1.0 + jnp.exp(-z))
            return c

        lax.fori_loop(0, SPAN // L, strip, 0)
        pltpu.sync_copy(cb_v, out_h.at[pl.ds(nb, SPAN)])

    return body(partials, x1t, params)


def kernel(x, edge_index, edge_attr,
           lin1_W, lin1_b, root1, bias1,
           lin2_W, lin2_b, root2, bias2):
    ei = edge_index.astype(jnp.int32)
    src = ei[0]
    dst = ei[1]
    attr_flat = edge_attr.T.reshape(-1)
    x_pad = jnp.pad(x, (0, NPAD - N))
    def _rows(v):
        return jnp.broadcast_to(v[:, None], (v.shape[0], L)).reshape(-1)

    p1 = _rows(jnp.concatenate([lin1_W.reshape(-1), lin1_b]))
    p2 = _rows(jnp.concatenate([root1.reshape(-1), bias1]))
    p3 = _rows(jnp.concatenate([lin2_W.reshape(-1), lin2_b]))
    p4 = _rows(jnp.concatenate([root2.reshape(-1), bias2]))
    parts1 = _edge_pass(x_pad, src, dst, attr_flat, p1, grouped_table=False)
    x1t = _build_x1(parts1, x_pad, p2)
    parts2 = _edge_pass(x1t, src, dst, attr_flat, p3, grouped_table=True)
    x2 = _build_x2(parts2, x1t, p4)
    return x2[:N].reshape(N, 1)


# speedup vs baseline: 29.9537x; 1.1834x over previous
import functools

import jax
import jax.numpy as jnp
from jax import lax
from jax.experimental import pallas as pl
from jax.experimental.pallas import tpu as pltpu
from jax.experimental.pallas import tpu_sc as plsc

NC, NS, L = 2, 16, 16
NW = NC * NS
N = 50000
NPAD = 50176
SPAN = NPAD // NW
E = 1600000
GROUPS = 4
TPG = NW // GROUPS
EPT = E // TPG
C = 2000
NCHUNK = EPT // C


def _mesh():
    return plsc.VectorSubcoreMesh(core_axis_name="c", subcore_axis_name="s")


def _wid():
    return lax.axis_index("s") * NC + lax.axis_index("c")


def _prow(par_v, row):
    return par_v[pl.ds(row * L, L)]


def _edge_pass(table, src, dst, attr_flat, params, grouped_table):
    @functools.partial(
        pl.kernel,
        mesh=_mesh(),
        compiler_params=pltpu.CompilerParams(needs_layout_passes=False),
        out_type=jax.ShapeDtypeStruct((NW * NPAD,), jnp.float32),
        scratch_types=[
            pltpu.VMEM((NPAD,), jnp.float32),
            pltpu.VMEM((NPAD,), jnp.float32),
            pltpu.VMEM((2 * C,), jnp.int32),
            pltpu.VMEM((2 * C,), jnp.int32),
            pltpu.VMEM((2 * 3 * C,), jnp.float32),
            pltpu.VMEM((16 * L,), jnp.float32),
            pltpu.SemaphoreType.DMA,
            pltpu.SemaphoreType.DMA,
        ],
    )
    def body(table_h, src_h, dst_h, attr_h, par_h, out_h,
             tab_v, acc_v, src_v, dst_v, attr_v, par_v, sem0, sem1):
        wid = _wid()
        g = wid // TPG
        t = wid % TPG
        pltpu.sync_copy(par_h, par_v)
        if grouped_table:
            pltpu.sync_copy(table_h.at[pl.ds(g * NPAD, NPAD)], tab_v)
        else:
            pltpu.sync_copy(table_h, tab_v)

        zeros16 = jnp.zeros((L,), jnp.float32)

        def zero_body(i, c):
            acc_v[pl.ds(i * L, L)] = zeros16
            return c

        lax.fori_loop(0, NPAD // L, zero_body, 0)

        wk0 = _prow(par_v, 3 * g)
        wk1 = _prow(par_v, 3 * g + 1)
        wk2 = _prow(par_v, 3 * g + 2)
        bk = _prow(par_v, 12 + g)
        ebase = t * EPT
        sems = (sem0, sem1)

        def chunk_copies(ci, slot):
            base = ebase + ci * C
            so = slot * C
            ao = slot * 3 * C
            return [
                (src_h.at[pl.ds(base, C)], src_v.at[pl.ds(so, C)]),
                (dst_h.at[pl.ds(base, C)], dst_v.at[pl.ds(so, C)]),
                (attr_h.at[pl.ds(base, C)], attr_v.at[pl.ds(ao, C)]),
                (attr_h.at[pl.ds(E + base, C)], attr_v.at[pl.ds(ao + C, C)]),
                (attr_h.at[pl.ds(2 * E + base, C)],
                 attr_v.at[pl.ds(ao + 2 * C, C)]),
            ]

        def start(ci, slot):
            for s, d in chunk_copies(ci, slot):
                pltpu.async_copy(s, d, sems[slot])

        def wait(ci, slot):
            for s, d in chunk_copies(ci, slot):
                pltpu.make_async_copy(s, d, sems[slot]).wait()

        def compute(slot):
            so = slot * C
            ao = slot * 3 * C

            @plsc.parallel_loop(0, C // L, unroll=4)
            def grp_body(gi):
                off = gi * L
                a0 = attr_v[pl.ds(ao + off, L)]
                a1 = attr_v[pl.ds(ao + C + off, L)]
                a2 = attr_v[pl.ds(ao + 2 * C + off, L)]
                w = jnp.maximum(a0 * wk0 + a1 * wk1 + a2 * wk2 + bk, 0.0)
                sv = src_v[pl.ds(so + off, L)]
                dv = dst_v[pl.ds(so + off, L)]
                xs = plsc.load_gather(tab_v, [sv])
                plsc.addupdate_scatter(acc_v, [dv], xs * w)

        start(0, 0)
        start(1, 1)

        def pair_body(j, c):
            c0 = 2 * j
            wait(c0, 0)
            compute(0)
            start(c0 + 2, 0)
            wait(c0 + 1, 1)
            compute(1)
            start(c0 + 3, 1)
            return c

        lax.fori_loop(0, NCHUNK // 2 - 1, pair_body, 0)
        wait(NCHUNK - 2, 0)
        compute(0)
        wait(NCHUNK - 1, 1)
        compute(1)
        pltpu.sync_copy(acc_v, out_h.at[pl.ds(wid * NPAD, NPAD)])

    return body(table, src, dst, attr_flat, params)


def _build_x1(partials, x_pad, params):
    @functools.partial(
        pl.kernel,
        mesh=_mesh(),
        compiler_params=pltpu.CompilerParams(needs_layout_passes=False),
        out_type=jax.ShapeDtypeStruct((GROUPS * NPAD,), jnp.float32),
        scratch_types=[
            pltpu.VMEM((NW * SPAN,), jnp.float32),
            pltpu.VMEM((SPAN,), jnp.float32),
            pltpu.VMEM((GROUPS * SPAN,), jnp.float32),
            pltpu.VMEM((8 * L,), jnp.float32),
        ],
    )
    def body(par_h, x_h, prm_h, out_h, pbuf_v, xb_v, cb_v, prm_v):
        wid = _wid()
        nb = wid * SPAN
        pltpu.sync_copy(prm_h, prm_v)
        for tt in range(NW):
            pltpu.sync_copy(par_h.at[pl.ds(tt * NPAD + nb, SPAN)],
                            pbuf_v.at[pl.ds(tt * SPAN, SPAN)])
        pltpu.sync_copy(x_h.at[pl.ds(nb, SPAN)], xb_v)
        for o in range(GROUPS):
            ro = _prow(prm_v, o)
            bo = _prow(prm_v, 4 + o)

            def strip(j, c, o=o, ro=ro, bo=bo):
                off = j * L
                s = jnp.zeros((L,), jnp.float32)
                for tt in range(o * TPG, (o + 1) * TPG):
                    s = s + pbuf_v[pl.ds(tt * SPAN + off, L)]
                xv = xb_v[pl.ds(off, L)]
                cb_v[pl.ds(o * SPAN + off, L)] = jnp.maximum(s + xv * ro + bo, 0.0)
                return c

            lax.fori_loop(0, SPAN // L, strip, 0)
            pltpu.sync_copy(cb_v.at[pl.ds(o * SPAN, SPAN)],
                            out_h.at[pl.ds(o * NPAD + nb, SPAN)])

    return body(partials, x_pad, params)


def _build_x2(partials, x1t, params):
    @functools.partial(
        pl.kernel,
        mesh=_mesh(),
        compiler_params=pltpu.CompilerParams(needs_layout_passes=False),
        out_type=jax.ShapeDtypeStruct((NPAD,), jnp.float32),
        scratch_types=[
            pltpu.VMEM((NW * SPAN,), jnp.float32),
            pltpu.VMEM((GROUPS * SPAN,), jnp.float32),
            pltpu.VMEM((SPAN,), jnp.float32),
            pltpu.VMEM((5 * L,), jnp.float32),
        ],
    )
    def body(par_h, x1_h, prm_h, out_h, pbuf_v, x1b_v, cb_v, prm_v):
        wid = _wid()
        nb = wid * SPAN
        pltpu.sync_copy(prm_h, prm_v)
        for tt in range(NW):
            pltpu.sync_copy(par_h.at[pl.ds(tt * NPAD + nb, SPAN)],
                            pbuf_v.at[pl.ds(tt * SPAN, SPAN)])
        for i in range(GROUPS):
            pltpu.sync_copy(x1_h.at[pl.ds(i * NPAD + nb, SPAN)],
                            x1b_v.at[pl.ds(i * SPAN, SPAN)])
        rv = [_prow(prm_v, i) for i in range(GROUPS)]
        bv = _prow(prm_v, GROUPS)

        def strip(j, c):
            off = j * L
            s = jnp.zeros((L,), jnp.float32)
            for tt in range(NW):
                s = s + pbuf_v[pl.ds(tt * SPAN + off, L)]
            for i in range(GROUPS):
                s = s + x1b_v[pl.ds(i * SPAN + off, L)] * rv[i]
            z = s + bv
            cb_v[pl.ds(off, L)] = 1.0 / (1.0 + jnp.exp(-z))
            return c

        lax.fori_loop(0, SPAN // L, strip, 0)
        pltpu.sync_copy(cb_v, out_h.at[pl.ds(nb, SPAN)])

    return body(partials, x1t, params)


def kernel(x, edge_index, edge_attr,
           lin1_W, lin1_b, root1, bias1,
           lin2_W, lin2_b, root2, bias2):
    ei = edge_index.astype(jnp.int32)
    src = ei[0]
    dst = ei[1]
    attr_flat = edge_attr.T.reshape(-1)
    x_pad = jnp.pad(x, (0, NPAD - N))
    def _rows(v):
        return jnp.broadcast_to(v[:, None], (v.shape[0], L)).reshape(-1)

    p1 = _rows(jnp.concatenate([lin1_W.reshape(-1), lin1_b]))
    p2 = _rows(jnp.concatenate([root1.reshape(-1), bias1]))
    p3 = _rows(jnp.concatenate([lin2_W.reshape(-1), lin2_b]))
    p4 = _rows(jnp.concatenate([root2.reshape(-1), bias2]))
    parts1 = _edge_pass(x_pad, src, dst, attr_flat, p1, grouped_table=False)
    x1t = _build_x1(parts1, x_pad, p2)
    parts2 = _edge_pass(x1t, src, dst, attr_flat, p3, grouped_table=True)
    x2 = _build_x2(parts2, x1t, p4)
    return x2[:N].reshape(N, 1)


# speedup vs baseline: 48.4915x; 1.6189x over previous
import functools

import jax
import jax.numpy as jnp
from jax import lax
from jax.experimental import pallas as pl
from jax.experimental.pallas import tpu as pltpu
from jax.experimental.pallas import tpu_sc as plsc

NC, NS, L = 2, 16, 16
NW = NC * NS
N = 50000
NPAD = 50176
SPAN = NPAD // NW
E = 1600000
GROUPS = 4
TPG = NW // GROUPS
EPT = E // TPG
C = 2000
NCHUNK = EPT // C


def _mesh():
    return plsc.VectorSubcoreMesh(core_axis_name="c", subcore_axis_name="s")


def _wid():
    return lax.axis_index("s") * NC + lax.axis_index("c")


def _prow(par_v, row):
    return par_v[pl.ds(row * L, L)]


def _edge_pass(table, src, dst, attrs, params, grouped_table):
    @functools.partial(
        pl.kernel,
        mesh=_mesh(),
        compiler_params=pltpu.CompilerParams(needs_layout_passes=False),
        out_type=jax.ShapeDtypeStruct((NW * NPAD,), jnp.float32),
        scratch_types=[
            pltpu.VMEM((NPAD,), jnp.float32),
            pltpu.VMEM((NPAD,), jnp.float32),
            pltpu.VMEM((2 * C,), jnp.int32),
            pltpu.VMEM((2 * C,), jnp.int32),
            pltpu.VMEM((2 * 3 * C,), jnp.float32),
            pltpu.VMEM((16 * L,), jnp.float32),
            pltpu.SemaphoreType.DMA,
            pltpu.SemaphoreType.DMA,
        ],
    )
    def body(table_h, src_h, dst_h, a0_h, a1_h, a2_h, par_h, out_h,
             tab_v, acc_v, src_v, dst_v, attr_v, par_v, sem0, sem1):
        wid = _wid()
        g = wid // TPG
        t = wid % TPG
        pltpu.sync_copy(par_h, par_v)
        if grouped_table:
            pltpu.sync_copy(table_h.at[pl.ds(g * NPAD, NPAD)], tab_v)
        else:
            pltpu.sync_copy(table_h, tab_v)

        zeros16 = jnp.zeros((L,), jnp.float32)

        def zero_body(i, c):
            acc_v[pl.ds(i * L, L)] = zeros16
            return c

        lax.fori_loop(0, NPAD // L, zero_body, 0)

        wk0 = _prow(par_v, 3 * g)
        wk1 = _prow(par_v, 3 * g + 1)
        wk2 = _prow(par_v, 3 * g + 2)
        bk = _prow(par_v, 12 + g)
        ebase = t * EPT
        sems = (sem0, sem1)

        def chunk_copies(ci, slot):
            base = ebase + ci * C
            so = slot * C
            ao = slot * 3 * C
            return [
                (src_h.at[pl.ds(base, C)], src_v.at[pl.ds(so, C)]),
                (dst_h.at[pl.ds(base, C)], dst_v.at[pl.ds(so, C)]),
                (a0_h.at[pl.ds(base, C)], attr_v.at[pl.ds(ao, C)]),
                (a1_h.at[pl.ds(base, C)], attr_v.at[pl.ds(ao + C, C)]),
                (a2_h.at[pl.ds(base, C)], attr_v.at[pl.ds(ao + 2 * C, C)]),
            ]

        def start(ci, slot):
            for s, d in chunk_copies(ci, slot):
                pltpu.async_copy(s, d, sems[slot])

        def wait(ci, slot):
            for s, d in chunk_copies(ci, slot):
                pltpu.make_async_copy(s, d, sems[slot]).wait()

        def compute(slot):
            so = slot * C
            ao = slot * 3 * C

            @plsc.parallel_loop(0, C // L, unroll=4)
            def grp_body(gi):
                off = gi * L
                a0 = attr_v[pl.ds(ao + off, L)]
                a1 = attr_v[pl.ds(ao + C + off, L)]
                a2 = attr_v[pl.ds(ao + 2 * C + off, L)]
                w = jnp.maximum(a0 * wk0 + a1 * wk1 + a2 * wk2 + bk, 0.0)
                sv = src_v[pl.ds(so + off, L)]
                dv = dst_v[pl.ds(so + off, L)]
                xs = plsc.load_gather(tab_v, [sv])
                plsc.addupdate_scatter(acc_v, [dv], xs * w)

        start(0, 0)
        start(1, 1)

        def pair_body(j, c):
            c0 = 2 * j
            wait(c0, 0)
            compute(0)
            start(c0 + 2, 0)
            wait(c0 + 1, 1)
            compute(1)
            start(c0 + 3, 1)
            return c

        lax.fori_loop(0, NCHUNK // 2 - 1, pair_body, 0)
        wait(NCHUNK - 2, 0)
        compute(0)
        wait(NCHUNK - 1, 1)
        compute(1)
        pltpu.sync_copy(acc_v, out_h.at[pl.ds(wid * NPAD, NPAD)])

    return body(table, src, dst, attrs[0], attrs[1], attrs[2], params)


def _build_x1(partials, x_pad, params):
    @functools.partial(
        pl.kernel,
        mesh=_mesh(),
        compiler_params=pltpu.CompilerParams(needs_layout_passes=False),
        out_type=jax.ShapeDtypeStruct((GROUPS * NPAD,), jnp.float32),
        scratch_types=[
            pltpu.VMEM((NW * SPAN,), jnp.float32),
            pltpu.VMEM((SPAN,), jnp.float32),
            pltpu.VMEM((GROUPS * SPAN,), jnp.float32),
            pltpu.VMEM((8 * L,), jnp.float32),
        ],
    )
    def body(par_h, x_h, prm_h, out_h, pbuf_v, xb_v, cb_v, prm_v):
        wid = _wid()
        nb = wid * SPAN
        pltpu.sync_copy(prm_h, prm_v)
        for tt in range(NW):
            pltpu.sync_copy(par_h.at[pl.ds(tt * NPAD + nb, SPAN)],
                            pbuf_v.at[pl.ds(tt * SPAN, SPAN)])
        pltpu.sync_copy(x_h.at[pl.ds(nb, SPAN)], xb_v)
        for o in range(GROUPS):
            ro = _prow(prm_v, o)
            bo = _prow(prm_v, 4 + o)

            def strip(j, c, o=o, ro=ro, bo=bo):
                off = j * L
                s = jnp.zeros((L,), jnp.float32)
                for tt in range(o * TPG, (o + 1) * TPG):
                    s = s + pbuf_v[pl.ds(tt * SPAN + off, L)]
                xv = xb_v[pl.ds(off, L)]
                cb_v[pl.ds(o * SPAN + off, L)] = jnp.maximum(s + xv * ro + bo, 0.0)
                return c

            lax.fori_loop(0, SPAN // L, strip, 0)
            pltpu.sync_copy(cb_v.at[pl.ds(o * SPAN, SPAN)],
                            out_h.at[pl.ds(o * NPAD + nb, SPAN)])

    return body(partials, x_pad, params)


def _build_x2(partials, x1t, params):
    @functools.partial(
        pl.kernel,
        mesh=_mesh(),
        compiler_params=pltpu.CompilerParams(needs_layout_passes=False),
        out_type=jax.ShapeDtypeStruct((NPAD,), jnp.float32),
        scratch_types=[
            pltpu.VMEM((NW * SPAN,), jnp.float32),
            pltpu.VMEM((GROUPS * SPAN,), jnp.float32),
            pltpu.VMEM((SPAN,), jnp.float32),
            pltpu.VMEM((5 * L,), jnp.float32),
        ],
    )
    def body(par_h, x1_h, prm_h, out_h, pbuf_v, x1b_v, cb_v, prm_v):
        wid = _wid()
        nb = wid * SPAN
        pltpu.sync_copy(prm_h, prm_v)
        for tt in range(NW):
            pltpu.sync_copy(par_h.at[pl.ds(tt * NPAD + nb, SPAN)],
                            pbuf_v.at[pl.ds(tt * SPAN, SPAN)])
        for i in range(GROUPS):
            pltpu.sync_copy(x1_h.at[pl.ds(i * NPAD + nb, SPAN)],
                            x1b_v.at[pl.ds(i * SPAN, SPAN)])
        rv = [_prow(prm_v, i) for i in range(GROUPS)]
        bv = _prow(prm_v, GROUPS)

        def strip(j, c):
            off = j * L
            s = jnp.zeros((L,), jnp.float32)
            for tt in range(NW):
                s = s + pbuf_v[pl.ds(tt * SPAN + off, L)]
            for i in range(GROUPS):
                s = s + x1b_v[pl.ds(i * SPAN + off, L)] * rv[i]
            z = s + bv
            cb_v[pl.ds(off, L)] = 1.0 / (1.0 + jnp.exp(-z))
            return c

        lax.fori_loop(0, SPAN // L, strip, 0)
        pltpu.sync_copy(cb_v, out_h.at[pl.ds(nb, SPAN)])

    return body(partials, x1t, params)


def kernel(x, edge_index, edge_attr,
           lin1_W, lin1_b, root1, bias1,
           lin2_W, lin2_b, root2, bias2):
    ei = edge_index.astype(jnp.int32)
    src = ei[0]
    dst = ei[1]
    attrs = (edge_attr[:, 0], edge_attr[:, 1], edge_attr[:, 2])
    x_pad = jnp.pad(x, (0, NPAD - N))
    def _rows(v):
        return jnp.broadcast_to(v[:, None], (v.shape[0], L)).reshape(-1)

    p1 = _rows(jnp.concatenate([lin1_W.reshape(-1), lin1_b]))
    p2 = _rows(jnp.concatenate([root1.reshape(-1), bias1]))
    p3 = _rows(jnp.concatenate([lin2_W.reshape(-1), lin2_b]))
    p4 = _rows(jnp.concatenate([root2.reshape(-1), bias2]))
    parts1 = _edge_pass(x_pad, src, dst, attrs, p1, grouped_table=False)
    x1t = _build_x1(parts1, x_pad, p2)
    parts2 = _edge_pass(x1t, src, dst, attrs, p3, grouped_table=True)
    x2 = _build_x2(parts2, x1t, p4)
    return x2[:N].reshape(N, 1)


# speedup vs baseline: 52.9319x; 1.0916x over previous
import functools

import jax
import jax.numpy as jnp
from jax import lax
from jax.experimental import pallas as pl
from jax.experimental.pallas import tpu as pltpu
from jax.experimental.pallas import tpu_sc as plsc

NC, NS, L = 2, 16, 16
NW = NC * NS
N = 50000
NPAD = 50176
SPAN = NPAD // NW
E = 1600000
GROUPS = 4
TPG = NW // GROUPS
EPT = E // TPG
C = 2000
NCHUNK = EPT // C


def _mesh():
    return plsc.VectorSubcoreMesh(core_axis_name="c", subcore_axis_name="s")


def _wid():
    return lax.axis_index("s") * NC + lax.axis_index("c")


def _prow(par_v, row):
    return par_v[pl.ds(row * L, L)]


def _edge_pass(table, src, dst, attrs, params, grouped_table):
    @functools.partial(
        pl.kernel,
        mesh=_mesh(),
        compiler_params=pltpu.CompilerParams(needs_layout_passes=False),
        out_type=jax.ShapeDtypeStruct((NW * NPAD,), jnp.float32),
        scratch_types=[
            pltpu.VMEM((NPAD,), jnp.float32),
            pltpu.VMEM((NPAD,), jnp.float32),
            pltpu.VMEM((2 * C,), jnp.int32),
            pltpu.VMEM((2 * C,), jnp.int32),
            pltpu.VMEM((2 * 3 * C,), jnp.float32),
            pltpu.VMEM((16 * L,), jnp.float32),
            pltpu.SemaphoreType.DMA,
            pltpu.SemaphoreType.DMA,
        ],
    )
    def body(table_h, src_h, dst_h, a0_h, a1_h, a2_h, par_h, out_h,
             tab_v, acc_v, src_v, dst_v, attr_v, par_v, sem0, sem1):
        wid = _wid()
        g = wid // TPG
        t = wid % TPG
        pltpu.sync_copy(par_h, par_v)
        if grouped_table:
            pltpu.sync_copy(table_h.at[pl.ds(g * NPAD, NPAD)], tab_v)
        else:
            pltpu.sync_copy(table_h, tab_v)

        zeros16 = jnp.zeros((L,), jnp.float32)

        def zero_body(i, c):
            acc_v[pl.ds(i * L, L)] = zeros16
            return c

        lax.fori_loop(0, NPAD // L, zero_body, 0)

        wk0 = _prow(par_v, 3 * g)
        wk1 = _prow(par_v, 3 * g + 1)
        wk2 = _prow(par_v, 3 * g + 2)
        bk = _prow(par_v, 12 + g)
        ebase = t * EPT
        sems = (sem0, sem1)

        def chunk_copies(ci, slot):
            base = ebase + ci * C
            so = slot * C
            ao = slot * 3 * C
            return [
                (src_h.at[pl.ds(base, C)], src_v.at[pl.ds(so, C)]),
                (dst_h.at[pl.ds(base, C)], dst_v.at[pl.ds(so, C)]),
                (a0_h.at[pl.ds(base, C)], attr_v.at[pl.ds(ao, C)]),
                (a1_h.at[pl.ds(base, C)], attr_v.at[pl.ds(ao + C, C)]),
                (a2_h.at[pl.ds(base, C)], attr_v.at[pl.ds(ao + 2 * C, C)]),
            ]

        def start(ci, slot):
            for s, d in chunk_copies(ci, slot):
                pltpu.async_copy(s, d, sems[slot])

        def wait(ci, slot):
            for s, d in chunk_copies(ci, slot):
                pltpu.make_async_copy(s, d, sems[slot]).wait()

        def compute(slot):
            so = slot * C
            ao = slot * 3 * C

            @plsc.parallel_loop(0, C // L, unroll=4)
            def grp_body(gi):
                off = gi * L
                a0 = attr_v[pl.ds(ao + off, L)]
                a1 = attr_v[pl.ds(ao + C + off, L)]
                a2 = attr_v[pl.ds(ao + 2 * C + off, L)]
                w = jnp.maximum(a0 * wk0 + a1 * wk1 + a2 * wk2 + bk, 0.0)
                sv = src_v[pl.ds(so + off, L)]
                dv = dst_v[pl.ds(so + off, L)]
                xs = plsc.load_gather(tab_v, [sv])
                plsc.addupdate_scatter(acc_v, [dv], xs * w)

        start(0, 0)
        start(1, 1)

        def pair_body(j, c):
            c0 = 2 * j
            wait(c0, 0)
            compute(0)
            start(c0 + 2, 0)
            wait(c0 + 1, 1)
            compute(1)
            start(c0 + 3, 1)
            return c

        lax.fori_loop(0, NCHUNK // 2 - 1, pair_body, 0)
        wait(NCHUNK - 2, 0)
        compute(0)
        wait(NCHUNK - 1, 1)
        compute(1)
        pltpu.sync_copy(acc_v, out_h.at[pl.ds(wid * NPAD, NPAD)])

    return body(table, src, dst, attrs[0], attrs[1], attrs[2], params)


def _build_x1(partials, x_pad, params):
    @functools.partial(
        pl.kernel,
        mesh=_mesh(),
        compiler_params=pltpu.CompilerParams(needs_layout_passes=False),
        out_type=jax.ShapeDtypeStruct((GROUPS * NPAD,), jnp.float32),
        scratch_types=[
            pltpu.VMEM((NW * SPAN,), jnp.float32),
            pltpu.VMEM((SPAN,), jnp.float32),
            pltpu.VMEM((GROUPS * SPAN,), jnp.float32),
            pltpu.VMEM((8 * L,), jnp.float32),
            pltpu.SemaphoreType.DMA,
        ],
    )
    def body(par_h, x_h, prm_h, out_h, pbuf_v, xb_v, cb_v, prm_v, sem):
        wid = _wid()
        nb = wid * SPAN
        stage = [(prm_h, prm_v), (x_h.at[pl.ds(nb, SPAN)], xb_v)]
        stage += [(par_h.at[pl.ds(tt * NPAD + nb, SPAN)],
                   pbuf_v.at[pl.ds(tt * SPAN, SPAN)]) for tt in range(NW)]
        for s, dd in stage:
            pltpu.async_copy(s, dd, sem)
        for s, dd in stage:
            pltpu.make_async_copy(s, dd, sem).wait()
        for o in range(GROUPS):
            ro = _prow(prm_v, o)
            bo = _prow(prm_v, 4 + o)

            def strip(j, c, o=o, ro=ro, bo=bo):
                off = j * L
                s = jnp.zeros((L,), jnp.float32)
                for tt in range(o * TPG, (o + 1) * TPG):
                    s = s + pbuf_v[pl.ds(tt * SPAN + off, L)]
                xv = xb_v[pl.ds(off, L)]
                cb_v[pl.ds(o * SPAN + off, L)] = jnp.maximum(s + xv * ro + bo, 0.0)
                return c

            lax.fori_loop(0, SPAN // L, strip, 0)
            pltpu.sync_copy(cb_v.at[pl.ds(o * SPAN, SPAN)],
                            out_h.at[pl.ds(o * NPAD + nb, SPAN)])

    return body(partials, x_pad, params)


def _build_x2(partials, x1t, params):
    @functools.partial(
        pl.kernel,
        mesh=_mesh(),
        compiler_params=pltpu.CompilerParams(needs_layout_passes=False),
        out_type=jax.ShapeDtypeStruct((NPAD,), jnp.float32),
        scratch_types=[
            pltpu.VMEM((NW * SPAN,), jnp.float32),
            pltpu.VMEM((GROUPS * SPAN,), jnp.float32),
            pltpu.VMEM((SPAN,), jnp.float32),
            pltpu.VMEM((5 * L,), jnp.float32),
            pltpu.SemaphoreType.DMA,
        ],
    )
    def body(par_h, x1_h, prm_h, out_h, pbuf_v, x1b_v, cb_v, prm_v, sem):
        wid = _wid()
        nb = wid * SPAN
        stage = [(prm_h, prm_v)]
        stage += [(par_h.at[pl.ds(tt * NPAD + nb, SPAN)],
                   pbuf_v.at[pl.ds(tt * SPAN, SPAN)]) for tt in range(NW)]
        stage += [(x1_h.at[pl.ds(i * NPAD + nb, SPAN)],
                   x1b_v.at[pl.ds(i * SPAN, SPAN)]) for i in range(GROUPS)]
        for s, dd in stage:
            pltpu.async_copy(s, dd, sem)
        for s, dd in stage:
            pltpu.make_async_copy(s, dd, sem).wait()
        rv = [_prow(prm_v, i) for i in range(GROUPS)]
        bv = _prow(prm_v, GROUPS)

        def strip(j, c):
            off = j * L
            s = jnp.zeros((L,), jnp.float32)
            for tt in range(NW):
                s = s + pbuf_v[pl.ds(tt * SPAN + off, L)]
            for i in range(GROUPS):
                s = s + x1b_v[pl.ds(i * SPAN + off, L)] * rv[i]
            z = s + bv
            cb_v[pl.ds(off, L)] = 1.0 / (1.0 + jnp.exp(-z))
            return c

        lax.fori_loop(0, SPAN // L, strip, 0)
        pltpu.sync_copy(cb_v, out_h.at[pl.ds(nb, SPAN)])

    return body(partials, x1t, params)


def kernel(x, edge_index, edge_attr,
           lin1_W, lin1_b, root1, bias1,
           lin2_W, lin2_b, root2, bias2):
    ei = edge_index.astype(jnp.int32)
    src = ei[0]
    dst = ei[1]
    attrs = (edge_attr[:, 0], edge_attr[:, 1], edge_attr[:, 2])
    x_pad = jnp.pad(x, (0, NPAD - N))
    def _rows(v):
        return jnp.broadcast_to(v[:, None], (v.shape[0], L)).reshape(-1)

    p1 = _rows(jnp.concatenate([lin1_W.reshape(-1), lin1_b]))
    p2 = _rows(jnp.concatenate([root1.reshape(-1), bias1]))
    p3 = _rows(jnp.concatenate([lin2_W.reshape(-1), lin2_b]))
    p4 = _rows(jnp.concatenate([root2.reshape(-1), bias2]))
    parts1 = _edge_pass(x_pad, src, dst, attrs, p1, grouped_table=False)
    x1t = _build_x1(parts1, x_pad, p2)
    parts2 = _edge_pass(x1t, src, dst, attrs, p3, grouped_table=True)
    x2 = _build_x2(parts2, x1t, p4)
    return x2[:N].reshape(N, 1)
